# baseline (speedup 1.0000x reference)
import jax
import jax.numpy as jnp
from jax.experimental import pallas as pl
from jax.experimental.pallas import tpu as pltpu

_CH = 1000
_NBUF = 4
_FL = _CH * 1024


def _probe_kernel(x_ref, s_ref, d_ref, i_ref, xbuf, in_sems):
    n1 = x_ref.shape[0]
    nchunk = n1 // _FL
    c = pl.program_id(0)

    def in_copy(k, slot):
        return pltpu.make_async_copy(
            x_ref.at[pl.ds(k * _FL, _FL)], xbuf.at[slot], in_sems.at[slot])

    @pl.when(c == 0)
    def _():
        for s in range(_NBUF):
            in_copy(s, s).start()

    slot = jax.lax.rem(c, _NBUF)
    in_copy(c, slot).wait()
    s_ref[...] = jnp.zeros(s_ref.shape, jnp.float32)
    d_ref[...] = jnp.zeros(d_ref.shape, jnp.float32)
    i_ref[...] = jnp.zeros(i_ref.shape, jnp.float32)

    @pl.when(c + _NBUF < nchunk)
    def _():
        in_copy(c + _NBUF, slot).start()


def kernel(x, W_cls, b_cls, W_bbox, b_bbox, W_iou, b_iou):
    n, d = x.shape
    kc = W_cls.shape[0]
    kb = W_bbox.shape[0]
    ki = W_iou.shape[0]
    x1 = x.reshape(-1)
    nchunk = n // _CH
    row_block = lambda i: (i, 0)
    scores, deltas, iou = pl.pallas_call(
        _probe_kernel,
        grid=(nchunk,),
        in_specs=[pl.BlockSpec(memory_space=pl.ANY)],
        out_specs=[
            pl.BlockSpec((_CH, kc), row_block),
            pl.BlockSpec((_CH, kb), row_block),
            pl.BlockSpec((_CH, ki), row_block),
        ],
        out_shape=[
            jax.ShapeDtypeStruct((n, kc), jnp.float32),
            jax.ShapeDtypeStruct((n, kb), jnp.float32),
            jax.ShapeDtypeStruct((n, ki), jnp.float32),
        ],
        scratch_shapes=[
            pltpu.VMEM((_NBUF, _FL), jnp.float32),
            pltpu.SemaphoreType.DMA((_NBUF,)),
        ],
        compiler_params=pltpu.CompilerParams(
            vmem_limit_bytes=110 * 1024 * 1024,
        ),
    )(x1)
    return scores, deltas, iou


# speedup vs baseline: 1.6744x; 1.6744x over previous
import jax
import jax.numpy as jnp
from jax.experimental import pallas as pl
from jax.experimental.pallas import tpu as pltpu

_CH = 2000
_NBUF = 4
_KP = 512
_OFF_CLS = 0
_OFF_BBOX = 128
_OFF_IOU = 448


def _heads_kernel(x_ref, w_ref, bc_ref, bb_ref, bi_ref,
                  s_ref, d_ref, i_ref,
                  xbuf, in_sems):
    n = x_ref.shape[0]
    nchunk = n // _CH
    kc = s_ref.shape[1]
    kb = d_ref.shape[1]
    ki = i_ref.shape[1]
    c = pl.program_id(0)

    def in_copy(k, slot):
        return pltpu.make_async_copy(
            x_ref.at[pl.ds(k * _CH, _CH), :], xbuf.at[slot], in_sems.at[slot])

    @pl.when(c == 0)
    def _():
        for s in range(_NBUF):
            in_copy(s, s).start()

    slot = jax.lax.rem(c, _NBUF)
    in_copy(c, slot).wait()
    y = jnp.dot(xbuf[slot].astype(jnp.bfloat16), w_ref[...],
                preferred_element_type=jnp.float32)
    s_ref[...] = y[:, _OFF_CLS:_OFF_CLS + kc] + bc_ref[...]
    d_ref[...] = y[:, _OFF_BBOX:_OFF_BBOX + kb] + bb_ref[...]
    i_ref[...] = y[:, _OFF_IOU:_OFF_IOU + ki] + bi_ref[...]

    @pl.when(c + _NBUF < nchunk)
    def _():
        in_copy(c + _NBUF, slot).start()


def kernel(x, W_cls, b_cls, W_bbox, b_bbox, W_iou, b_iou):
    if x.ndim > 2:
        x = x.reshape(x.shape[0], -1)
    n, d = x.shape
    kc = W_cls.shape[0]
    kb = W_bbox.shape[0]
    ki = W_iou.shape[0]

    w = jnp.concatenate([
        W_cls.T, jnp.zeros((d, _OFF_BBOX - kc), jnp.float32),
        W_bbox.T, W_iou.T,
        jnp.zeros((d, _KP - _OFF_IOU - ki), jnp.float32),
    ], axis=1).astype(jnp.bfloat16)
    bc = b_cls.reshape(1, kc)
    bb = b_bbox.reshape(1, kb)
    bi = b_iou.reshape(1, ki)

    nchunk = n // _CH
    row_block = lambda i: (i, 0)
    whole = lambda i: (0, 0)

    scores, deltas, iou = pl.pallas_call(
        _heads_kernel,
        grid=(nchunk,),
        in_specs=[
            pl.BlockSpec(memory_space=pl.ANY),
            pl.BlockSpec((d, _KP), whole),
            pl.BlockSpec((1, kc), whole),
            pl.BlockSpec((1, kb), whole),
            pl.BlockSpec((1, ki), whole),
        ],
        out_specs=[
            pl.BlockSpec((_CH, kc), row_block),
            pl.BlockSpec((_CH, kb), row_block),
            pl.BlockSpec((_CH, ki), row_block),
        ],
        out_shape=[
            jax.ShapeDtypeStruct((n, kc), jnp.float32),
            jax.ShapeDtypeStruct((n, kb), jnp.float32),
            jax.ShapeDtypeStruct((n, ki), jnp.float32),
        ],
        scratch_shapes=[
            pltpu.VMEM((_NBUF, _CH, d), jnp.float32),
            pltpu.SemaphoreType.DMA((_NBUF,)),
        ],
        compiler_params=pltpu.CompilerParams(
            vmem_limit_bytes=110 * 1024 * 1024,
            needs_layout_passes=False,
        ),
    )(x, w, bc, bb, bi)
    return scores, deltas, iou


# speedup vs baseline: 1.6811x; 1.0040x over previous
import jax
import jax.numpy as jnp
from jax.experimental import pallas as pl
from jax.experimental.pallas import tpu as pltpu

_CH = 1000
_NBUF = 6
_KP = 512
_OFF_CLS = 0
_OFF_BBOX = 128
_OFF_IOU = 448


def _heads_kernel(x_ref, w_ref, bc_ref, bb_ref, bi_ref,
                  s_ref, d_ref, i_ref,
                  xbuf, in_sems):
    n = x_ref.shape[0]
    nchunk = n // _CH
    kc = s_ref.shape[1]
    kb = d_ref.shape[1]
    ki = i_ref.shape[1]
    c = pl.program_id(0)

    def in_copy(k, slot):
        return pltpu.make_async_copy(
            x_ref.at[pl.ds(k * _CH, _CH), :], xbuf.at[slot], in_sems.at[slot])

    @pl.when(c == 0)
    def _():
        for s in range(_NBUF):
            in_copy(s, s).start()

    slot = jax.lax.rem(c, _NBUF)
    in_copy(c, slot).wait()
    y = jnp.dot(xbuf[slot].astype(jnp.bfloat16), w_ref[...],
                preferred_element_type=jnp.float32)
    s_ref[...] = y[:, _OFF_CLS:_OFF_CLS + kc] + bc_ref[...]
    d_ref[...] = y[:, _OFF_BBOX:_OFF_BBOX + kb] + bb_ref[...]
    i_ref[...] = y[:, _OFF_IOU:_OFF_IOU + ki] + bi_ref[...]

    @pl.when(c + _NBUF < nchunk)
    def _():
        in_copy(c + _NBUF, slot).start()


def kernel(x, W_cls, b_cls, W_bbox, b_bbox, W_iou, b_iou):
    if x.ndim > 2:
        x = x.reshape(x.shape[0], -1)
    n, d = x.shape
    kc = W_cls.shape[0]
    kb = W_bbox.shape[0]
    ki = W_iou.shape[0]

    w = jnp.concatenate([
        W_cls.T, jnp.zeros((d, _OFF_BBOX - kc), jnp.float32),
        W_bbox.T, W_iou.T,
        jnp.zeros((d, _KP - _OFF_IOU - ki), jnp.float32),
    ], axis=1).astype(jnp.bfloat16)
    bc = b_cls.reshape(1, kc)
    bb = b_bbox.reshape(1, kb)
    bi = b_iou.reshape(1, ki)

    nchunk = n // _CH
    row_block = lambda i: (i, 0)
    whole = lambda i: (0, 0)

    scores, deltas, iou = pl.pallas_call(
        _heads_kernel,
        grid=(nchunk,),
        in_specs=[
            pl.BlockSpec(memory_space=pl.ANY),
            pl.BlockSpec((d, _KP), whole),
            pl.BlockSpec((1, kc), whole),
            pl.BlockSpec((1, kb), whole),
            pl.BlockSpec((1, ki), whole),
        ],
        out_specs=[
            pl.BlockSpec((_CH, kc), row_block),
            pl.BlockSpec((_CH, kb), row_block),
            pl.BlockSpec((_CH, ki), row_block),
        ],
        out_shape=[
            jax.ShapeDtypeStruct((n, kc), jnp.float32),
            jax.ShapeDtypeStruct((n, kb), jnp.float32),
            jax.ShapeDtypeStruct((n, ki), jnp.float32),
        ],
        scratch_shapes=[
            pltpu.VMEM((_NBUF, _CH, d), jnp.float32),
            pltpu.SemaphoreType.DMA((_NBUF,)),
        ],
        compiler_params=pltpu.CompilerParams(
            vmem_limit_bytes=110 * 1024 * 1024,
            needs_layout_passes=False,
        ),
    )(x, w, bc, bb, bi)
    return scores, deltas, iou


# speedup vs baseline: 1.6830x; 1.0011x over previous
import jax
import jax.numpy as jnp
from jax.experimental import pallas as pl
from jax.experimental.pallas import tpu as pltpu

_CH = 1000
_NBUF = 4
_KP = 512
_OFF_CLS = 0
_OFF_BBOX = 128
_OFF_IOU = 448


def _heads_kernel(x_ref, w_ref, bc_ref, bb_ref, bi_ref,
                  s_ref, d_ref, i_ref,
                  xbuf, in_sems):
    n = x_ref.shape[0]
    nchunk = n // _CH
    kc = s_ref.shape[1]
    kb = d_ref.shape[1]
    ki = i_ref.shape[1]
    c = pl.program_id(0)

    def in_copy(k, slot):
        return pltpu.make_async_copy(
            x_ref.at[pl.ds(k * _CH, _CH), :], xbuf.at[slot], in_sems.at[slot])

    @pl.when(c == 0)
    def _():
        for s in range(_NBUF):
            in_copy(s, s).start()

    slot = jax.lax.rem(c, _NBUF)
    in_copy(c, slot).wait()
    y = jnp.dot(xbuf[slot].astype(jnp.bfloat16), w_ref[...],
                preferred_element_type=jnp.float32)
    s_ref[...] = y[:, _OFF_CLS:_OFF_CLS + kc] + bc_ref[...]
    d_ref[...] = y[:, _OFF_BBOX:_OFF_BBOX + kb] + bb_ref[...]
    i_ref[...] = y[:, _OFF_IOU:_OFF_IOU + ki] + bi_ref[...]

    @pl.when(c + _NBUF < nchunk)
    def _():
        in_copy(c + _NBUF, slot).start()


def kernel(x, W_cls, b_cls, W_bbox, b_bbox, W_iou, b_iou):
    if x.ndim > 2:
        x = x.reshape(x.shape[0], -1)
    n, d = x.shape
    kc = W_cls.shape[0]
    kb = W_bbox.shape[0]
    ki = W_iou.shape[0]

    w = jnp.concatenate([
        W_cls.T, jnp.zeros((d, _OFF_BBOX - kc), jnp.float32),
        W_bbox.T, W_iou.T,
        jnp.zeros((d, _KP - _OFF_IOU - ki), jnp.float32),
    ], axis=1).astype(jnp.bfloat16)
    bc = b_cls.reshape(1, kc)
    bb = b_bbox.reshape(1, kb)
    bi = b_iou.reshape(1, ki)

    nchunk = n // _CH
    row_block = lambda i: (i, 0)
    whole = lambda i: (0, 0)

    scores, deltas, iou = pl.pallas_call(
        _heads_kernel,
        grid=(nchunk,),
        in_specs=[
            pl.BlockSpec(memory_space=pl.ANY),
            pl.BlockSpec((d, _KP), whole),
            pl.BlockSpec((1, kc), whole),
            pl.BlockSpec((1, kb), whole),
            pl.BlockSpec((1, ki), whole),
        ],
        out_specs=[
            pl.BlockSpec((_CH, kc), row_block),
            pl.BlockSpec((_CH, kb), row_block),
            pl.BlockSpec((_CH, ki), row_block),
        ],
        out_shape=[
            jax.ShapeDtypeStruct((n, kc), jnp.float32),
            jax.ShapeDtypeStruct((n, kb), jnp.float32),
            jax.ShapeDtypeStruct((n, ki), jnp.float32),
        ],
        scratch_shapes=[
            pltpu.VMEM((_NBUF, _CH, d), jnp.float32),
            pltpu.SemaphoreType.DMA((_NBUF,)),
        ],
        compiler_params=pltpu.CompilerParams(
            vmem_limit_bytes=110 * 1024 * 1024,
            needs_layout_passes=False,
        ),
    )(x, w, bc, bb, bi)
    return scores, deltas, iou
